# baseline (speedup 1.0000x reference)
import functools

import jax
import jax.numpy as jnp
from jax import lax
from jax.experimental import pallas as pl
from jax.experimental.pallas import tpu as pltpu
from jax.experimental.pallas import tpu_sc as plsc

_EMBED = 64
_SCALE = 8.0
_V = 23
_L = 16

_NJT = 25
_NIT = 128
_NC = 4
_CT = _NIT // _NC


def _prep_body(x_ref, t_ref, o1_ref, o2_ref):
    for it in range(_NIT):
        o1_ref[0, it] = x_ref[0, :, pl.ds(it * 128, 128)]
    t8 = t_ref[...] * _SCALE
    row = jnp.concatenate([t8, jnp.zeros((_V, 64), jnp.float32)], axis=1)
    o2_ref[...] = jnp.concatenate([row, jnp.zeros((1, 128), jnp.float32)], axis=0)


def _prep():
    return pl.pallas_call(
        _prep_body,
        grid=(_NJT,),
        in_specs=[
            pl.BlockSpec((1, 8, 16384), lambda j: (0, j, 0)),
            pl.BlockSpec((_V, _EMBED), lambda j: (0, 0)),
        ],
        out_specs=[
            pl.BlockSpec((1, _NIT, 8, 128), lambda j: (j, 0, 0, 0)),
            pl.BlockSpec((_V + 1, 128), lambda j: (0, 0)),
        ],
        out_shape=[
            jax.ShapeDtypeStruct((_NJT, _NIT, 8, 128), jnp.int32),
            jax.ShapeDtypeStruct((_V + 1, 128), jnp.float32),
        ],
    )


def _sc_fill():
    def body(idx_hbm, tpk_hbm, out_hbm, tpv, idxb, outb, osem0, osem1):
        wid = lax.axis_index("s") * 2 + lax.axis_index("c")
        kt = wid // _NC
        c = wid % _NC
        it0 = c * _CT
        pltpu.sync_copy(tpk_hbm, tpv)
        kvecs = [jnp.zeros((_L,), jnp.int32) + (kt * 8 + kk) for kk in range(8)]
        osems = [osem0, osem1]

        def per_jt(jt, carry):
            pltpu.sync_copy(idx_hbm.at[jt, pl.ds(it0, _CT)], idxb)
            for jj in range(8):
                p = jj % 2
                j = jt * 8 + jj

                def _wait():
                    pltpu.make_async_copy(
                        out_hbm.at[j, kt, pl.ds(it0, _CT)],
                        outb.at[p],
                        osems[p],
                    ).wait()

                if jj >= 2:
                    _wait()
                else:
                    pl.when(jt > 0)(_wait)

                def per_it(it, c2):
                    for g in range(8):
                        idx16 = idxb[it, jj, pl.ds(g * _L, _L)]
                        for kk in range(8):
                            v = plsc.load_gather(tpv, [idx16, kvecs[kk]])
                            outb[p, it, kk, pl.ds(g * _L, _L)] = v
                    return c2

                lax.fori_loop(0, _CT, per_it, 0)
                pltpu.async_copy(
                    outb.at[p], out_hbm.at[j, kt, pl.ds(it0, _CT)], osems[p]
                )
            return carry

        lax.fori_loop(0, _NJT, per_jt, 0)
        for p in range(2):
            pltpu.make_async_copy(
                out_hbm.at[0, kt, pl.ds(it0, _CT)], outb.at[p], osems[p]
            ).wait()

    return pl.kernel(
        body,
        out_type=jax.ShapeDtypeStruct((200, 8, _NIT, 8, 128), jnp.float32),
        mesh=plsc.VectorSubcoreMesh(core_axis_name="c", subcore_axis_name="s"),
        compiler_params=pltpu.CompilerParams(needs_layout_passes=False),
        scratch_types=[
            pltpu.VMEM((_V + 1, 128), jnp.float32),
            pltpu.VMEM((_CT, 8, 128), jnp.int32),
            pltpu.VMEM((2, _CT, 8, 128), jnp.float32),
            pltpu.SemaphoreType.DMA,
            pltpu.SemaphoreType.DMA,
        ],
    )


@functools.cache
def _kernels():
    return _prep(), _sc_fill()


def kernel(x, table):
    b, t, _ = x.shape
    xt = jnp.transpose(x, (2, 1, 0))
    prep, fill = _kernels()
    idx5, tpk = prep(xt, table)
    out5 = fill(idx5, tpk)
    outp = jnp.transpose(out5, (2, 4, 0, 1, 3))
    return outp.reshape(b, t, _EMBED)


# speedup vs baseline: 19.6725x; 19.6725x over previous
import functools

import jax
import jax.numpy as jnp
from jax import lax
from jax.experimental import pallas as pl
from jax.experimental.pallas import tpu as pltpu

_EMBED = 64
_SCALE = 8.0
_V = 23

_BJ = 8
_BL = 4096


def _onehot_body(x_ref, t_ref, o_ref):
    t8 = t_ref[...] * _SCALE
    vals = lax.broadcasted_iota(jnp.int32, (_V, _BL), 0)
    for jj in range(_BJ):
        idx = x_ref[0, jj, :]
        oh = (idx[None, :] == vals).astype(jnp.float32)
        o_ref[jj] = lax.dot_general(
            t8, oh, (((0,), (0,)), ((), ())),
            preferred_element_type=jnp.float32,
        )


@functools.cache
def _lookup_kernel(nj, ni):
    grid = (nj // _BJ, ni // _BL)
    return pl.pallas_call(
        _onehot_body,
        grid=grid,
        in_specs=[
            pl.BlockSpec((1, _BJ, _BL), lambda j, i: (0, j, i)),
            pl.BlockSpec((_V, _EMBED), lambda j, i: (0, 0)),
        ],
        out_specs=pl.BlockSpec((_BJ, _EMBED, _BL), lambda j, i: (j, 0, i)),
        out_shape=jax.ShapeDtypeStruct((nj, _EMBED, ni), jnp.float32),
    )


def kernel(x, table):
    b, t, _ = x.shape
    xt = jnp.transpose(x, (2, 1, 0))
    outt = _lookup_kernel(t, b)(xt, table)
    return jnp.transpose(outt, (2, 0, 1))


# speedup vs baseline: 19.7271x; 1.0028x over previous
import functools

import jax
import jax.numpy as jnp
from jax import lax
from jax.experimental import pallas as pl
from jax.experimental.pallas import tpu as pltpu

_EMBED = 64
_SCALE = 8.0
_V = 23

_BJ = 8
_BL = 8192


def _onehot_body(x_ref, t_ref, o_ref):
    t8 = t_ref[...] * _SCALE
    vals = lax.broadcasted_iota(jnp.int32, (_V, _BL), 0)
    for jj in range(_BJ):
        idx = x_ref[0, jj, :]
        oh = (idx[None, :] == vals).astype(jnp.float32)
        o_ref[jj] = lax.dot_general(
            t8, oh, (((0,), (0,)), ((), ())),
            preferred_element_type=jnp.float32,
        )


@functools.cache
def _lookup_kernel(nj, ni):
    grid = (nj // _BJ, ni // _BL)
    return pl.pallas_call(
        _onehot_body,
        grid=grid,
        in_specs=[
            pl.BlockSpec((1, _BJ, _BL), lambda j, i: (0, j, i)),
            pl.BlockSpec((_V, _EMBED), lambda j, i: (0, 0)),
        ],
        out_specs=pl.BlockSpec((_BJ, _EMBED, _BL), lambda j, i: (j, 0, i)),
        out_shape=jax.ShapeDtypeStruct((nj, _EMBED, ni), jnp.float32),
    )


def kernel(x, table):
    b, t, _ = x.shape
    xt = jnp.transpose(x, (2, 1, 0))
    outt = _lookup_kernel(t, b)(xt, table)
    return jnp.transpose(outt, (2, 0, 1))
